# Initial kernel scaffold; baseline (speedup 1.0000x reference)
import functools

import jax
import jax.numpy as jnp
from jax import lax
from jax.experimental import pallas as pl
from jax.experimental.pallas import tpu as pltpu
from jax.experimental.pallas import tpu_sc as plsc

VOCAB = 100000
EMBED_DIM = 128
BATCH = 16384
NUM_FIELDS = 100

NC = 2
NS = 16
NW = NC * NS
NB = BATCH // NW
K = 4
NCHUNK = NB // K
LANES = 16
DC = EMBED_DIM // LANES
WPAD = 112


def _fm_kernel(idx_hbm, table_hbm, w_hbm, bias_hbm, out_hbm,
               idx_v, rows_v, w_v, out_v, bias_v, sem):
    wid = lax.axis_index("s") * NC + lax.axis_index("c")
    base = wid * NB

    pltpu.sync_copy(bias_hbm, bias_v)
    for j in range(K):
        w_v[j, pl.ds(WPAD - LANES, LANES)] = jnp.zeros((LANES,), jnp.float32)
    bias_s = bias_v[0]

    def chunk(g, carry):
        b0 = base + g * K
        pltpu.sync_copy(idx_hbm.at[pl.ds(b0, K)], idx_v)
        copies = []
        for j in range(K):
            idx_row = idx_v.at[j]
            copies.append(pltpu.async_copy(table_hbm.at[idx_row],
                                           rows_v.at[j], sem))
            copies.append(pltpu.async_copy(w_hbm.at[idx_row],
                                           w_v.at[j, pl.ds(0, NUM_FIELDS)],
                                           sem))
        for c in copies:
            c.wait()

        for j in range(K):
            zero = jnp.zeros((LANES,), jnp.float32)

            def fbody(f, carry):
                svecs, qv = carry
                new_s = []
                for dc in range(DC):
                    e = rows_v[j, f, pl.ds(dc * LANES, LANES)]
                    new_s.append(svecs[dc] + e)
                    qv = qv + e * e
                return (tuple(new_s), qv)

            (svecs, qv) = lax.fori_loop(
                0, NUM_FIELDS, fbody,
                (tuple(zero for _ in range(DC)), zero))

            fmv = zero
            for dc in range(DC):
                fmv = fmv + svecs[dc] * svecs[dc]
            lw = zero
            for c in range(WPAD // LANES):
                lw = lw + w_v[j, pl.ds(c * LANES, LANES)]
            val = (jnp.sum(lw) + bias_s
                   + 0.5 * (jnp.sum(fmv) - jnp.sum(qv)))
            out_v[g * K + j] = val
        return carry

    lax.fori_loop(0, NCHUNK, chunk, 0)
    pltpu.sync_copy(out_v, out_hbm.at[pl.ds(base, NB)])


def kernel(interaction_pairs, emb_table, linear_weight, linear_bias):
    idx = interaction_pairs.astype(jnp.int32)
    w_flat = linear_weight.reshape(-1)
    mesh = plsc.VectorSubcoreMesh(core_axis_name="c", subcore_axis_name="s")
    fm = functools.partial(
        pl.kernel,
        mesh=mesh,
        out_type=jax.ShapeDtypeStruct((BATCH,), jnp.float32),
        scratch_types=[
            pltpu.VMEM((K, NUM_FIELDS), jnp.int32),
            pltpu.VMEM((K, NUM_FIELDS, EMBED_DIM), jnp.float32),
            pltpu.VMEM((K, WPAD), jnp.float32),
            pltpu.VMEM((NB,), jnp.float32),
            pltpu.VMEM((1,), jnp.float32),
            pltpu.SemaphoreType.DMA,
        ],
    )(_fm_kernel)
    return fm(idx, emb_table, w_flat, linear_bias)


# speedup vs baseline: 7.8384x; 7.8384x over previous
import functools

import jax
import jax.numpy as jnp
from jax import lax
from jax.experimental import pallas as pl
from jax.experimental.pallas import tpu as pltpu
from jax.experimental.pallas import tpu_sc as plsc

VOCAB = 100000
EMBED_DIM = 128
BATCH = 16384
NUM_FIELDS = 100

NC = 2
NS = 16
NW = NC * NS
NB = BATCH // NW
K = 4
NCHUNK = NB // K
LANES = 16
DC = EMBED_DIM // LANES
FPAD = 104


def _fm_kernel(idx_hbm, table_hbm, w_hbm, bias_hbm, out_hbm,
               idx_v, rows_v, w_v, out_v, bias_v, sem):
    wid = lax.axis_index("s") * NC + lax.axis_index("c")
    base = wid * NB

    pltpu.sync_copy(bias_hbm, bias_v)
    bias_s = bias_v[pl.ds(0, LANES)][0]
    lane = lax.iota(jnp.int32, LANES)

    def chunk(g, carry):
        b0 = base + g * K
        pltpu.sync_copy(idx_hbm.at[pl.ds(b0, K)], idx_v)
        copies = []
        for j in range(K):
            copies.append(pltpu.async_copy(table_hbm.at[idx_v.at[j]],
                                           rows_v.at[j], sem))
            copies.append(pltpu.async_copy(w_hbm.at[idx_v.at[j]],
                                           w_v.at[j], sem))
        for c in copies:
            c.wait()

        for j in range(K):
            zero = jnp.zeros((LANES,), jnp.float32)

            def fbody(f, carry):
                svecs, qv = carry
                new_s = []
                for dc in range(DC):
                    e = rows_v[j, f, pl.ds(dc * LANES, LANES)]
                    new_s.append(svecs[dc] + e)
                    qv = qv + e * e
                return (tuple(new_s), qv)

            (svecs, qv) = lax.fori_loop(
                0, NUM_FIELDS, fbody,
                (tuple(zero for _ in range(DC)), zero))

            lw = zero
            for c in range(6):
                lw = lw + w_v[j, pl.ds(c * LANES, LANES)]
            tmask = lane < (NUM_FIELDS - 6 * LANES)
            tidx = jnp.where(tmask, 6 * LANES + lane, 0)
            tail = plsc.load_gather(w_v, [jnp.full((LANES,), j, jnp.int32),
                                          tidx], mask=tmask)
            lw = lw + jnp.where(tmask, tail, 0.0)

            fmv = zero
            for dc in range(DC):
                fmv = fmv + svecs[dc] * svecs[dc]
            cv = lw + 0.5 * (fmv - qv)
            val = jnp.sum(cv) + bias_s
            plsc.store_scatter(out_v,
                               [jnp.full((LANES,), g * K + j, jnp.int32)],
                               jnp.full((LANES,), val, jnp.float32),
                               mask=lane == 0)
        return carry

    lax.fori_loop(0, NCHUNK, chunk, 0)
    pltpu.sync_copy(out_v, out_hbm.at[pl.ds(base, NB)])


def kernel(interaction_pairs, emb_table, linear_weight, linear_bias):
    idx = jnp.pad(interaction_pairs.astype(jnp.int32),
                  ((0, 0), (0, FPAD - NUM_FIELDS)))
    w_flat = linear_weight.reshape(-1)
    bias_pad = jnp.pad(linear_bias.astype(jnp.float32), (0, LANES - 1))
    mesh = plsc.VectorSubcoreMesh(core_axis_name="c", subcore_axis_name="s")
    fm = functools.partial(
        pl.kernel,
        mesh=mesh,
        compiler_params=pltpu.CompilerParams(needs_layout_passes=False,
                                             use_tc_tiling_on_sc=False),
        out_type=jax.ShapeDtypeStruct((BATCH,), jnp.float32),
        scratch_types=[
            pltpu.VMEM((K, FPAD), jnp.int32),
            pltpu.VMEM((K, FPAD, EMBED_DIM), jnp.float32),
            pltpu.VMEM((K, FPAD), jnp.float32),
            pltpu.VMEM((NB,), jnp.float32),
            pltpu.VMEM((LANES,), jnp.float32),
            pltpu.SemaphoreType.DMA,
        ],
    )(_fm_kernel)
    return fm(idx, emb_table, w_flat, bias_pad)
